# baseline (speedup 1.0000x reference)
import functools

import jax
import jax.numpy as jnp
from jax import lax
from jax.experimental import pallas as pl
from jax.experimental.pallas import tpu as pltpu
from jax.experimental.pallas import tpu_sc as plsc

B = 16384
C = 1000
CP = 1024
_TL = CP // 128

_BLK = 512


def _softmax_body(pred_ref, out_ref):
    x = pred_ref[...]
    m = jnp.max(x, axis=-1, keepdims=True)
    e = jnp.exp(x - m)
    z = jnp.sum(e, axis=-1, keepdims=True)
    p = jnp.pad(e / z, ((0, 0), (0, CP - C)))
    out_ref[...] = jnp.reshape(p, (_BLK, _TL, 128))


def _tc_softmax(predictions):
    rows = predictions.shape[0]
    return pl.pallas_call(
        _softmax_body,
        grid=(rows // _BLK,),
        in_specs=[pl.BlockSpec((_BLK, C), lambda i: (i, 0))],
        out_specs=pl.BlockSpec((_BLK, _TL, 128), lambda i: (i, 0, 0)),
        out_shape=jax.ShapeDtypeStruct((rows, _TL, 128), jnp.float32),
    )(predictions)


_NC = 2
_NS = 16
_NW = _NC * _NS
_B_PER_W = B // _NW
_CHUNK = 16
_N_CHUNKS = _B_PER_W // _CHUNK
_SROWS = 1000


def _sc_segment_sum(probs, idx):
    mesh = plsc.VectorSubcoreMesh(core_axis_name="c", subcore_axis_name="s")
    rows = probs.shape[0]
    b_per_w = rows // _NW
    n_chunks = b_per_w // _CHUNK

    @functools.partial(
        pl.kernel,
        mesh=mesh,
        out_type=jax.ShapeDtypeStruct((_NC * _SROWS, _TL, 128), jnp.float32),
        compiler_params=pltpu.CompilerParams(use_tc_tiling_on_sc=False),
        scratch_types=[
            pltpu.VMEM_SHARED((_SROWS, _TL, 128), jnp.float32),
            pltpu.VMEM((_CHUNK,), jnp.int32),
            pltpu.VMEM((_CHUNK, _TL, 128), jnp.float32),
            pltpu.VMEM((_CHUNK, _TL, 128), jnp.float32),
            pltpu.SemaphoreType.DMA,
            pltpu.SemaphoreType.DMA,
        ],
    )
    def k(probs_hbm, idx_hbm, out_hbm, acc_sh, idx16, buf0, buf1, sem0, sem1):
        c = lax.axis_index("c")
        s = lax.axis_index("s")
        wid = s * _NC + c
        base = wid * b_per_w

        zv = jnp.zeros((16,), jnp.float32)

        def zrow(kk, _):
            for r in range(_CHUNK):
                for t in range(_TL):
                    buf0[r, t, pl.ds(kk * 16, 16)] = zv
            return ()

        lax.fori_loop(0, 128 // 16, zrow, (), unroll=False)

        @pl.when(s < _NS - 1)
        def _():
            for h in range(4):
                pltpu.sync_copy(buf0, acc_sh.at[pl.ds(s * 64 + h * _CHUNK,
                                                      _CHUNK)])

        @pl.when(s == _NS - 1)
        def _():
            for h in range(2):
                pltpu.sync_copy(buf0, acc_sh.at[pl.ds(960 + h * _CHUNK,
                                                      _CHUNK)])
            pltpu.sync_copy(buf0.at[pl.ds(0, 8)], acc_sh.at[pl.ds(992, 8)])

        plsc.subcore_barrier()

        bufs = (buf0, buf1)
        sems = (sem0, sem1)
        copies = [None, None]
        copies[0] = pltpu.async_copy(
            probs_hbm.at[pl.ds(base, _CHUNK)], buf0, sem0)
        for j in range(n_chunks):
            cur = j % 2
            nxt = 1 - cur
            if j + 1 < n_chunks:
                copies[nxt] = pltpu.async_copy(
                    probs_hbm.at[pl.ds(base + (j + 1) * _CHUNK, _CHUNK)],
                    bufs[nxt], sems[nxt])
            pltpu.sync_copy(idx_hbm.at[pl.ds(base + j * _CHUNK, _CHUNK)],
                            idx16)
            copies[cur].wait()
            pltpu.sync_copy(bufs[cur], acc_sh.at[idx16], add=True)
        plsc.subcore_barrier()

        @pl.when(s < _NS - 1)
        def _():
            pltpu.sync_copy(acc_sh.at[pl.ds(s * 64, 64)],
                            out_hbm.at[pl.ds(c * _SROWS + s * 64, 64)])

        @pl.when(s == _NS - 1)
        def _():
            pltpu.sync_copy(acc_sh.at[pl.ds(960, 40)],
                            out_hbm.at[pl.ds(c * _SROWS + 960, 40)])

    return k(probs, idx)


_RBLK = 200


def _reduce_body(s0_ref, s1_ref, s2_ref, s3_ref, cm_ref, out_ref):
    i = pl.program_id(0)
    s3 = (s0_ref[...] + s1_ref[...]) + (s2_ref[...] + s3_ref[...])
    s = jnp.reshape(s3, (_RBLK, CP))
    row = i * _RBLK + lax.broadcasted_iota(jnp.int32, (_RBLK, 1), 0)
    cm_p = jnp.pad(cm_ref[...], ((0, 0), (0, CP - C)))
    lane = lax.broadcasted_iota(jnp.int32, (_RBLK, CP), 1)
    diag = jnp.sum(jnp.where(lane == row, cm_p, 0.0), axis=-1, keepdims=True)
    blk = jnp.sum(s * (cm_p - diag))

    @pl.when(i == 0)
    def _():
        out_ref[...] = jnp.zeros_like(out_ref)

    out_ref[...] += blk


def _tc_reduce(sp0, sp1, cost_matrix):
    out = pl.pallas_call(
        _reduce_body,
        grid=(_SROWS // _RBLK,),
        in_specs=[
            pl.BlockSpec((_RBLK, _TL, 128), lambda i: (i, 0, 0)),
            pl.BlockSpec((_RBLK, _TL, 128),
                         lambda i: (i + _SROWS // _RBLK, 0, 0)),
            pl.BlockSpec((_RBLK, _TL, 128), lambda i: (i, 0, 0)),
            pl.BlockSpec((_RBLK, _TL, 128),
                         lambda i: (i + _SROWS // _RBLK, 0, 0)),
            pl.BlockSpec((_RBLK, C), lambda i: (i, 0)),
        ],
        out_specs=pl.BlockSpec((1, 1), lambda i: (0, 0)),
        out_shape=jax.ShapeDtypeStruct((1, 1), jnp.float32),
    )(sp0, sp0, sp1, sp1, cost_matrix)
    return out[0, 0]


def kernel(predictions, targets, cost_matrix):
    tgt = targets.astype(jnp.int32)
    h = B // 2
    p0 = _tc_softmax(predictions[:h])
    sp0 = _sc_segment_sum(p0, tgt[:h])
    p1 = _tc_softmax(predictions[h:])
    sp1 = _sc_segment_sum(p1, tgt[h:])
    total = _tc_reduce(sp0, sp1, cost_matrix)
    return total / jnp.float32(B)


# speedup vs baseline: 1.2756x; 1.2756x over previous
import functools

import jax
import jax.numpy as jnp
from jax import lax
from jax.experimental import pallas as pl
from jax.experimental.pallas import tpu as pltpu
from jax.experimental.pallas import tpu_sc as plsc

B = 16384
C = 1000
CP = 1024
_TL = CP // 128

_BLK = 1024


def _softmax_body(pred_ref, out_ref):
    x = pred_ref[...]
    m = jnp.max(x, axis=-1, keepdims=True)
    e = jnp.exp(x - m)
    z = jnp.sum(e, axis=-1, keepdims=True)
    p = jnp.pad(e / z, ((0, 0), (0, CP - C)))
    out_ref[...] = jnp.reshape(p, (_BLK, _TL, 128))


def _tc_softmax(predictions):
    return pl.pallas_call(
        _softmax_body,
        grid=(B // _BLK,),
        in_specs=[pl.BlockSpec((_BLK, C), lambda i: (i, 0))],
        out_specs=pl.BlockSpec((_BLK, _TL, 128), lambda i: (i, 0, 0)),
        out_shape=jax.ShapeDtypeStruct((B, _TL, 128), jnp.float32),
    )(predictions)


_NC = 2
_NS = 16
_NW = _NC * _NS
_B_PER_W = B // _NW
_CHUNK = 16
_N_CHUNKS = _B_PER_W // _CHUNK
_SROWS = 1000


def _sc_segment_sum(probs, idx):
    mesh = plsc.VectorSubcoreMesh(core_axis_name="c", subcore_axis_name="s")

    @functools.partial(
        pl.kernel,
        mesh=mesh,
        out_type=jax.ShapeDtypeStruct((_NC * _SROWS, _TL, 128), jnp.float32),
        compiler_params=pltpu.CompilerParams(use_tc_tiling_on_sc=False),
        scratch_types=[
            pltpu.VMEM_SHARED((_SROWS, _TL, 128), jnp.float32),
            pltpu.VMEM((_CHUNK,), jnp.int32),
            pltpu.VMEM((_CHUNK, _TL, 128), jnp.float32),
            pltpu.VMEM((_CHUNK, _TL, 128), jnp.float32),
            pltpu.SemaphoreType.DMA,
            pltpu.SemaphoreType.DMA,
        ],
    )
    def k(probs_hbm, idx_hbm, out_hbm, acc_sh, idx16, buf0, buf1, sem0, sem1):
        c = lax.axis_index("c")
        s = lax.axis_index("s")
        wid = s * _NC + c
        base = wid * _B_PER_W

        zv = jnp.zeros((16,), jnp.float32)

        def zrow(kk, _):
            for r in range(_CHUNK):
                for t in range(_TL):
                    buf0[r, t, pl.ds(kk * 16, 16)] = zv
            return ()

        lax.fori_loop(0, 128 // 16, zrow, (), unroll=False)

        @pl.when(s < _NS - 1)
        def _():
            for h in range(4):
                pltpu.sync_copy(buf0, acc_sh.at[pl.ds(s * 64 + h * _CHUNK,
                                                      _CHUNK)])

        @pl.when(s == _NS - 1)
        def _():
            for h in range(2):
                pltpu.sync_copy(buf0, acc_sh.at[pl.ds(960 + h * _CHUNK,
                                                      _CHUNK)])
            pltpu.sync_copy(buf0.at[pl.ds(0, 8)], acc_sh.at[pl.ds(992, 8)])

        plsc.subcore_barrier()

        bufs = (buf0, buf1)
        sems = (sem0, sem1)
        copies = [None, None]
        copies[0] = pltpu.async_copy(
            probs_hbm.at[pl.ds(base, _CHUNK)], buf0, sem0)
        for j in range(_N_CHUNKS):
            cur = j % 2
            nxt = 1 - cur
            if j + 1 < _N_CHUNKS:
                copies[nxt] = pltpu.async_copy(
                    probs_hbm.at[pl.ds(base + (j + 1) * _CHUNK, _CHUNK)],
                    bufs[nxt], sems[nxt])
            pltpu.sync_copy(idx_hbm.at[pl.ds(base + j * _CHUNK, _CHUNK)],
                            idx16)
            copies[cur].wait()
            pltpu.sync_copy(bufs[cur], acc_sh.at[idx16], add=True)
        plsc.subcore_barrier()

        @pl.when(s < _NS - 1)
        def _():
            pltpu.sync_copy(acc_sh.at[pl.ds(s * 64, 64)],
                            out_hbm.at[pl.ds(c * _SROWS + s * 64, 64)])

        @pl.when(s == _NS - 1)
        def _():
            pltpu.sync_copy(acc_sh.at[pl.ds(960, 40)],
                            out_hbm.at[pl.ds(c * _SROWS + 960, 40)])

    return k(probs, idx)


_RBLK = 200


def _reduce_body(s0_ref, s1_ref, cm_ref, out_ref):
    i = pl.program_id(0)
    s3 = s0_ref[...] + s1_ref[...]
    s = jnp.reshape(s3, (_RBLK, CP))
    row = i * _RBLK + lax.broadcasted_iota(jnp.int32, (_RBLK, 1), 0)
    cm_p = jnp.pad(cm_ref[...], ((0, 0), (0, CP - C)))
    lane = lax.broadcasted_iota(jnp.int32, (_RBLK, CP), 1)
    diag = jnp.sum(jnp.where(lane == row, cm_p, 0.0), axis=-1, keepdims=True)
    blk = jnp.sum(s * (cm_p - diag))

    @pl.when(i == 0)
    def _():
        out_ref[...] = jnp.zeros_like(out_ref)

    out_ref[...] += blk


def _tc_reduce(s_parts, cost_matrix):
    out = pl.pallas_call(
        _reduce_body,
        grid=(_SROWS // _RBLK,),
        in_specs=[
            pl.BlockSpec((_RBLK, _TL, 128), lambda i: (i, 0, 0)),
            pl.BlockSpec((_RBLK, _TL, 128),
                         lambda i: (i + _SROWS // _RBLK, 0, 0)),
            pl.BlockSpec((_RBLK, C), lambda i: (i, 0)),
        ],
        out_specs=pl.BlockSpec((1, 1), lambda i: (0, 0)),
        out_shape=jax.ShapeDtypeStruct((1, 1), jnp.float32),
    )(s_parts, s_parts, cost_matrix)
    return out[0, 0]


def kernel(predictions, targets, cost_matrix):
    probs = _tc_softmax(predictions)
    s_parts = _sc_segment_sum(probs, targets.astype(jnp.int32))
    total = _tc_reduce(s_parts, cost_matrix)
    return total / jnp.float32(B)
